# initial kernel scaffold (unmeasured)
import jax
import jax.numpy as jnp
from jax import lax
from jax.experimental import pallas as pl
from jax.experimental.pallas import tpu as pltpu


def kernel(
    x,
):
    def body(*refs):
        pass

    out_shape = jax.ShapeDtypeStruct(..., jnp.float32)
    return pl.pallas_call(body, out_shape=out_shape)(...)


# baseline (device time: 198143 ns/iter reference)
import jax
import jax.numpy as jnp
from jax import lax
from jax.experimental import pallas as pl
from jax.experimental.pallas import tpu as pltpu


def kernel(x):
    m, n = x.shape
    half = m // 2

    def body(x_ref, out_ref, ybuf, send_sems, recv_sems):
        my_x = lax.axis_index("x")
        my_y = lax.axis_index("y")

        barrier_sem = pltpu.get_barrier_semaphore()
        pl.semaphore_signal(
            barrier_sem, inc=1,
            device_id=(my_x, 1 - my_y), device_id_type=pl.DeviceIdType.MESH,
        )
        pl.semaphore_signal(
            barrier_sem, inc=1,
            device_id=(1 - my_x, my_y), device_id_type=pl.DeviceIdType.MESH,
        )
        pl.semaphore_wait(barrier_sem, 2)

        row0 = my_x * half

        rdma_y = pltpu.make_async_remote_copy(
            src_ref=x_ref.at[pl.ds(row0, half)],
            dst_ref=ybuf,
            send_sem=send_sems.at[0],
            recv_sem=recv_sems.at[0],
            device_id=(my_x, 1 - my_y),
            device_id_type=pl.DeviceIdType.MESH,
        )
        rdma_y.start()
        rdma_y.wait()

        out_ref[pl.ds(row0, half), :] = x_ref[pl.ds(row0, half), :] + ybuf[:, :]

        rdma_x = pltpu.make_async_remote_copy(
            src_ref=out_ref.at[pl.ds(row0, half)],
            dst_ref=out_ref.at[pl.ds(row0, half)],
            send_sem=send_sems.at[1],
            recv_sem=recv_sems.at[1],
            device_id=(1 - my_x, my_y),
            device_id_type=pl.DeviceIdType.MESH,
        )
        rdma_x.start()
        rdma_x.wait()

    return pl.pallas_call(
        body,
        out_shape=jax.ShapeDtypeStruct((m, n), x.dtype),
        in_specs=[pl.BlockSpec(memory_space=pltpu.VMEM)],
        out_specs=pl.BlockSpec(memory_space=pltpu.VMEM),
        scratch_shapes=[
            pltpu.VMEM((half, n), x.dtype),
            pltpu.SemaphoreType.DMA((2,)),
            pltpu.SemaphoreType.DMA((2,)),
        ],
        compiler_params=pltpu.CompilerParams(collective_id=0),
    )(x)


# device time: 113554 ns/iter; 1.7449x vs baseline; 1.7449x over previous
import jax
import jax.numpy as jnp
from jax import lax
from jax.experimental import pallas as pl
from jax.experimental.pallas import tpu as pltpu


N_CHUNKS = 16


def kernel(x):
    m, n = x.shape
    half = m // 2
    ch = half // N_CHUNKS

    def body(x_ref, out_ref, ybuf, y_send, y_recv, x_send, x_recv):
        my_x = lax.axis_index("x")
        my_y = lax.axis_index("y")

        barrier_sem = pltpu.get_barrier_semaphore()
        pl.semaphore_signal(
            barrier_sem, inc=1,
            device_id=(my_x, 1 - my_y), device_id_type=pl.DeviceIdType.MESH,
        )
        pl.semaphore_signal(
            barrier_sem, inc=1,
            device_id=(1 - my_x, my_y), device_id_type=pl.DeviceIdType.MESH,
        )
        pl.semaphore_wait(barrier_sem, 2)

        row0 = my_x * half

        y_rdmas = []
        for c in range(N_CHUNKS):
            r = pltpu.make_async_remote_copy(
                src_ref=x_ref.at[pl.ds(row0 + c * ch, ch)],
                dst_ref=ybuf.at[pl.ds(c * ch, ch)],
                send_sem=y_send.at[c],
                recv_sem=y_recv.at[c],
                device_id=(my_x, 1 - my_y),
                device_id_type=pl.DeviceIdType.MESH,
            )
            r.start()
            y_rdmas.append(r)

        x_rdmas = []
        for c in range(N_CHUNKS):
            y_rdmas[c].wait_recv()
            out_ref[pl.ds(row0 + c * ch, ch), :] = (
                x_ref[pl.ds(row0 + c * ch, ch), :] + ybuf[pl.ds(c * ch, ch), :]
            )
            r = pltpu.make_async_remote_copy(
                src_ref=out_ref.at[pl.ds(row0 + c * ch, ch)],
                dst_ref=out_ref.at[pl.ds(row0 + c * ch, ch)],
                send_sem=x_send.at[c],
                recv_sem=x_recv.at[c],
                device_id=(1 - my_x, my_y),
                device_id_type=pl.DeviceIdType.MESH,
            )
            r.start()
            x_rdmas.append(r)

        for c in range(N_CHUNKS):
            y_rdmas[c].wait_send()
            x_rdmas[c].wait()

    return pl.pallas_call(
        body,
        out_shape=jax.ShapeDtypeStruct((m, n), x.dtype),
        in_specs=[pl.BlockSpec(memory_space=pltpu.VMEM)],
        out_specs=pl.BlockSpec(memory_space=pltpu.VMEM),
        scratch_shapes=[
            pltpu.VMEM((half, n), x.dtype),
            pltpu.SemaphoreType.DMA((N_CHUNKS,)),
            pltpu.SemaphoreType.DMA((N_CHUNKS,)),
            pltpu.SemaphoreType.DMA((N_CHUNKS,)),
            pltpu.SemaphoreType.DMA((N_CHUNKS,)),
        ],
        compiler_params=pltpu.CompilerParams(collective_id=0),
    )(x)


# device time: 111681 ns/iter; 1.7742x vs baseline; 1.0168x over previous
import jax
import jax.numpy as jnp
from jax import lax
from jax.experimental import pallas as pl
from jax.experimental.pallas import tpu as pltpu


N_CHUNKS = 32


def kernel(x):
    m, n = x.shape
    half = m // 2
    ch = half // N_CHUNKS

    def body(x_ref, out_ref, ybuf, y_send, y_recv, x_send, x_recv):
        my_x = lax.axis_index("x")
        my_y = lax.axis_index("y")

        barrier_sem = pltpu.get_barrier_semaphore()
        pl.semaphore_signal(
            barrier_sem, inc=1,
            device_id=(my_x, 1 - my_y), device_id_type=pl.DeviceIdType.MESH,
        )
        pl.semaphore_signal(
            barrier_sem, inc=1,
            device_id=(1 - my_x, my_y), device_id_type=pl.DeviceIdType.MESH,
        )
        pl.semaphore_wait(barrier_sem, 2)

        row0 = my_x * half

        y_rdmas = []
        for c in range(N_CHUNKS):
            r = pltpu.make_async_remote_copy(
                src_ref=x_ref.at[pl.ds(row0 + c * ch, ch)],
                dst_ref=ybuf.at[pl.ds(c * ch, ch)],
                send_sem=y_send.at[c],
                recv_sem=y_recv.at[c],
                device_id=(my_x, 1 - my_y),
                device_id_type=pl.DeviceIdType.MESH,
            )
            r.start()
            y_rdmas.append(r)

        x_rdmas = []
        for c in range(N_CHUNKS):
            y_rdmas[c].wait_recv()
            out_ref[pl.ds(row0 + c * ch, ch), :] = (
                x_ref[pl.ds(row0 + c * ch, ch), :] + ybuf[pl.ds(c * ch, ch), :]
            )
            r = pltpu.make_async_remote_copy(
                src_ref=out_ref.at[pl.ds(row0 + c * ch, ch)],
                dst_ref=out_ref.at[pl.ds(row0 + c * ch, ch)],
                send_sem=x_send.at[c],
                recv_sem=x_recv.at[c],
                device_id=(1 - my_x, my_y),
                device_id_type=pl.DeviceIdType.MESH,
            )
            r.start()
            x_rdmas.append(r)

        for c in range(N_CHUNKS):
            y_rdmas[c].wait_send()
            x_rdmas[c].wait()

    return pl.pallas_call(
        body,
        out_shape=jax.ShapeDtypeStruct((m, n), x.dtype),
        in_specs=[pl.BlockSpec(memory_space=pltpu.VMEM)],
        out_specs=pl.BlockSpec(memory_space=pltpu.VMEM),
        scratch_shapes=[
            pltpu.VMEM((half, n), x.dtype),
            pltpu.SemaphoreType.DMA((N_CHUNKS,)),
            pltpu.SemaphoreType.DMA((N_CHUNKS,)),
            pltpu.SemaphoreType.DMA((N_CHUNKS,)),
            pltpu.SemaphoreType.DMA((N_CHUNKS,)),
        ],
        compiler_params=pltpu.CompilerParams(collective_id=0),
    )(x)
